# baseline (speedup 1.0000x reference)
import functools

import jax
import jax.numpy as jnp
from jax import lax
from jax.experimental import pallas as pl
from jax.experimental.pallas import tpu as pltpu
from jax.experimental.pallas import tpu_sc as plsc

N = 2048
E = 131072
IN_DIM = 28
H = 64
OUT_DIM = 28
R = 4
HEADS = 4
HEAD_DIM = H // HEADS

HE = H + 16
NC = 2
NS = 16
NW = NC * NS
EPW = E // NW
CHUNK = 128
NCHUNK = EPW // CHUNK

_dot = functools.partial(
    lax.dot_general, precision=lax.Precision.HIGHEST,
    preferred_element_type=jnp.float32)


def _mm(a, b):
    return _dot(a, b, (((a.ndim - 1,), (0,)), ((), ())))


def _mm_t(a, b):
    return _dot(a, b, (((1,), (1,)), ((), ())))


def _leaky(x):
    return jnp.where(x >= 0, x, 0.2 * x)


def _proj_body(x_ref, w_in_ref, b_in_ref, w_tfc_ref, b_tfc_ref,
               wq_ref, wk_ref, wv_ref, bq_ref, bk_ref, bv_ref,
               q_ref, k_ref, v_ref):
    xh = _mm(x_ref[0], w_in_ref[...]) + b_in_ref[...]
    xh = _mm(xh, w_tfc_ref[...]) + b_tfc_ref[...]
    for h in range(HEADS):
        q_ref[h] = _mm_t(xh, wq_ref[h]) + bq_ref[h]
        k_ref[h] = _mm_t(xh, wk_ref[h]) + bk_ref[h]
        v_ref[h] = _mm_t(xh, wv_ref[h]) + bv_ref[h]


def _proj(x, w_in, b_in, w_tfc, b_tfc, wq, wk, wv, bq, bk, bv):
    hshape = jax.ShapeDtypeStruct((HEADS, N, HEAD_DIM), jnp.float32)
    return pl.pallas_call(
        _proj_body,
        out_shape=(hshape, hshape, hshape),
    )(x, w_in, b_in, w_tfc, b_tfc, wq, wk, wv, bq, bk, bv)


def _attn_body(q_ref, k_ref, v_ref, ow_ref, ob_ref, o_ref):
    h = pl.program_id(0)
    s = _mm_t(q_ref[0] * (1.0 / (HEAD_DIM ** 0.5)), k_ref[0])
    m = jnp.max(s, axis=1, keepdims=True)
    e = jnp.exp(s - m)
    p = e / jnp.sum(e, axis=1, keepdims=True)
    contrib = _mm(_mm(p, v_ref[0]), ow_ref[0])

    @pl.when(h == 0)
    def _():
        o_ref[...] = contrib + ob_ref[...]

    @pl.when(h != 0)
    def _():
        o_ref[...] = o_ref[...] + contrib


def _attn(q, k, v, ow, ob):
    spec = pl.BlockSpec((1, N, HEAD_DIM), lambda h: (h, 0, 0))
    return pl.pallas_call(
        _attn_body,
        grid=(HEADS,),
        in_specs=[spec, spec, spec,
                  pl.BlockSpec((1, HEAD_DIM, H), lambda h: (h, 0, 0)),
                  pl.BlockSpec((1, H), lambda h: (0, 0))],
        out_specs=pl.BlockSpec((N, H), lambda h: (0, 0)),
        out_shape=jax.ShapeDtypeStruct((N, H), jnp.float32),
    )(q, k, v, ow, ob)


_TABLES_OUT = (
    jax.ShapeDtypeStruct((R * N, HE), jnp.float32),
    jax.ShapeDtypeStruct((N, HE), jnp.float32),
    jax.ShapeDtypeStruct((N, 1), jnp.float32),
    jax.ShapeDtypeStruct((N, 1), jnp.float32),
    jax.ShapeDtypeStruct((1, 1), jnp.float32),
)


def _tables_core(xh, rw_ref, gw_ref, asrc_ref, adst_ref,
                 hr_ref, hx_ref, hs_ref, hd_ref, c_ref):
    onescol = jnp.concatenate(
        [jnp.ones((N, 1), jnp.float32), jnp.zeros((N, HE - H - 1), jnp.float32)],
        axis=1)
    for r in range(R):
        hr_ref[r * N:(r + 1) * N, 0:H] = _mm(xh, rw_ref[r])
        hr_ref[r * N:(r + 1) * N, H:HE] = onescol
    h = _mm(xh, gw_ref[...])
    hx_ref[:, 0:H] = h
    hx_ref[:, H:HE] = onescol
    hs = _mm(h, asrc_ref[...])
    hd = _mm(h, adst_ref[...])
    hs_ref[...] = hs
    hd_ref[...] = hd
    c = _leaky(jnp.max(hs) + jnp.max(hd))
    c_ref[...] = jnp.full((1, 1), c, jnp.float32)


def _combine_core(xh, rp_ref, gp_ref, root_ref, rb_ref, gb_ref,
                  hx_ref, hs_ref, hd_ref, c_ref):
    agg = jnp.zeros((N, H), jnp.float32)
    for r in range(R):
        blk = rp_ref[0, r * N:(r + 1) * N, :] + rp_ref[1, r * N:(r + 1) * N, :]
        cnt = jnp.maximum(blk[:, H:H + 1], 1.0)
        agg = agg + blk[:, 0:H] / cnt
    xr = agg + _mm(xh, root_ref[...]) + rb_ref[...]

    gp = gp_ref[0] + gp_ref[1]
    c = c_ref[0, 0]
    exn = jnp.exp(_leaky(hs_ref[...] + hd_ref[...]) - c)
    num = gp[:, 0:H] + exn * hx_ref[:, 0:H]
    den = gp[:, H:H + 1] + exn
    xg = num / den + gb_ref[...]
    return jnp.maximum(xr + xg, 0.0)


def _tables_body(xh_ref, rw_ref, gw_ref, asrc_ref, adst_ref,
                 hr_ref, hx_ref, hs_ref, hd_ref, c_ref):
    _tables_core(xh_ref[...], rw_ref, gw_ref, asrc_ref, adst_ref,
                 hr_ref, hx_ref, hs_ref, hd_ref, c_ref)


def _tables(xh, rw, gw, asrc, adst):
    return pl.pallas_call(
        _tables_body,
        out_shape=_TABLES_OUT,
    )(xh, rw, gw, asrc, adst)


def _combine_tables_body(xh_ref, rp_ref, gp_ref, root_ref, rb_ref, gb_ref,
                         hx_ref, hs_ref, hd_ref, c_ref,
                         rw2_ref, gw2_ref, asrc2_ref, adst2_ref,
                         xh2_ref, hr2_ref, hx2_ref, hs2_ref, hd2_ref, c2_ref):
    xh2 = _combine_core(xh_ref[...], rp_ref, gp_ref, root_ref, rb_ref, gb_ref,
                        hx_ref, hs_ref, hd_ref, c_ref)
    xh2_ref[...] = xh2
    _tables_core(xh2, rw2_ref, gw2_ref, asrc2_ref, adst2_ref,
                 hr2_ref, hx2_ref, hs2_ref, hd2_ref, c2_ref)


def _combine_tables(xh, rp, gp, root, rb, gb, hx, hs, hd, c,
                    rw2, gw2, asrc2, adst2):
    return pl.pallas_call(
        _combine_tables_body,
        out_shape=(jax.ShapeDtypeStruct((N, H), jnp.float32),) + _TABLES_OUT,
    )(xh, rp, gp, root, rb, gb, hx, hs, hd, c, rw2, gw2, asrc2, adst2)


def _combine_out_body(xh_ref, rp_ref, gp_ref, root_ref, rb_ref, gb_ref,
                      hx_ref, hs_ref, hd_ref, c_ref, wout_ref, bout_ref,
                      out_ref):
    xh2 = _combine_core(xh_ref[...], rp_ref, gp_ref, root_ref, rb_ref, gb_ref,
                        hx_ref, hs_ref, hd_ref, c_ref)
    out_ref[...] = _mm(xh2, wout_ref[...]) + bout_ref[...]


def _combine_out(xh, rp, gp, root, rb, gb, hx, hs, hd, c, wout, bout):
    return pl.pallas_call(
        _combine_out_body,
        out_shape=jax.ShapeDtypeStruct((N, OUT_DIM), jnp.float32),
    )(xh, rp, gp, root, rb, gb, hx, hs, hd, c, wout, bout)


def _sc_edge_body(gidx_hbm, seg_hbm, src_hbm, dst_hbm, hr_hbm, hx_hbm,
                  hs_hbm, hd_hbm, c_hbm,
                  rgcn_out, gat_out,
                  rows0_v, rows1_v, sg0_v, sg1_v, ex_v, ia_v, ib_v,
                  ga_v, hs_v, hd_v, c_v,
                  racc, gacc, sem0, sem1):
    cid = lax.axis_index("c")
    sid = lax.axis_index("s")
    wid = cid * NS + sid
    base = wid * EPW
    rows = (rows0_v, rows1_v)
    sgs = (sg0_v, sg1_v)
    sems = (sem0, sem1)

    pltpu.sync_copy(src_hbm.at[pl.ds(base, EPW)], ia_v)
    pltpu.sync_copy(dst_hbm.at[pl.ds(base, EPW)], ib_v)
    pltpu.sync_copy(gidx_hbm.at[pl.ds(base, EPW)], ga_v)
    pltpu.sync_copy(hs_hbm, hs_v)
    pltpu.sync_copy(hd_hbm, hd_v)
    pltpu.sync_copy(c_hbm, c_v)

    def zrow(i, _):
        r = i // (HE // 16)
        k = i % (HE // 16)
        rows0_v[r, pl.ds(k * 16, 16)] = jnp.zeros((16,), jnp.float32)
        return 0
    lax.fori_loop(0, CHUNK * (HE // 16), zrow, 0)

    rrows = (R * N) // NS
    grows = N // NS
    for j in range(rrows // CHUNK):
        pltpu.sync_copy(rows0_v, racc.at[pl.ds(sid * rrows + j * CHUNK, CHUNK)])
    pltpu.sync_copy(rows0_v, gacc.at[pl.ds(sid * grows, grows)])
    plsc.subcore_barrier()

    c = c_v[...][0]

    def score(g, _):
        s16 = ia_v[pl.ds(g * 16, 16)]
        d16 = ib_v[pl.ds(g * 16, 16)]
        sc = plsc.load_gather(hs_v, [s16]) + plsc.load_gather(hd_v, [d16])
        ex_v[pl.ds(g * 16, 16)] = jnp.exp(_leaky(sc) - c)
        return 0
    lax.fori_loop(0, EPW // 16, score, 0)

    TOT = 2 * NCHUNK

    def issue(c, b):
        @pl.when(c < NCHUNK)
        def _():
            pltpu.sync_copy(dst_hbm.at[pl.ds(base + c * CHUNK, CHUNK)], sgs[b])
            pltpu.async_copy(hx_hbm.at[ia_v.at[pl.ds(c * CHUNK, CHUNK)]],
                             rows[b], sems[b])

        @pl.when(c >= NCHUNK)
        def _():
            cr = c - NCHUNK
            pltpu.sync_copy(seg_hbm.at[pl.ds(base + cr * CHUNK, CHUNK)], sgs[b])
            pltpu.async_copy(hr_hbm.at[ga_v.at[pl.ds(cr * CHUNK, CHUNK)]],
                             rows[b], sems[b])

    def drain_process(c, b):
        pltpu.make_async_copy(hx_hbm.at[ia_v.at[pl.ds(0, CHUNK)]],
                              rows[b], sems[b]).wait()

        @pl.when(c < NCHUNK)
        def _():
            def scalegrp(g, _):
                ex16 = ex_v[pl.ds(c * CHUNK + g * 16, 16)]
                for i in range(16):
                    a = ex16[i]
                    e = g * 16 + i
                    for j in range(HE // 16):
                        sl = pl.ds(j * 16, 16)
                        rows[b][e, sl] = rows[b][e, sl] * a
                return 0
            lax.fori_loop(0, CHUNK // 16, scalegrp, 0)
            pltpu.sync_copy(rows[b], gacc.at[sgs[b]], add=True)

        @pl.when(c >= NCHUNK)
        def _():
            pltpu.sync_copy(rows[b], racc.at[sgs[b]], add=True)

    pltpu.sync_copy(dst_hbm.at[pl.ds(base, CHUNK)], sg0_v)
    pltpu.async_copy(hx_hbm.at[ia_v.at[pl.ds(0, CHUNK)]], rows0_v, sem0)

    def pipe(i, _):
        for b in range(2):
            c = 2 * i + b

            @pl.when(c + 1 < TOT)
            def _():
                issue(c + 1, 1 - b)
            drain_process(c, b)
        return 0
    lax.fori_loop(0, TOT // 2, pipe, 0)

    plsc.subcore_barrier()
    for j in range(rrows // CHUNK):
        off = sid * rrows + j * CHUNK
        pltpu.sync_copy(racc.at[pl.ds(off, CHUNK)],
                        rgcn_out.at[cid, pl.ds(off, CHUNK)])
    pltpu.sync_copy(gacc.at[pl.ds(sid * grows, grows)],
                    gat_out.at[cid, pl.ds(sid * grows, grows)])


@functools.lru_cache(maxsize=1)
def _build_sc_edge():
    return pl.kernel(
        _sc_edge_body,
        out_type=(
            jax.ShapeDtypeStruct((NC, R * N, HE), jnp.float32),
            jax.ShapeDtypeStruct((NC, N, HE), jnp.float32),
        ),
        mesh=plsc.VectorSubcoreMesh(core_axis_name="c", subcore_axis_name="s",
                                    num_cores=NC, num_subcores=NS),
        compiler_params=pltpu.CompilerParams(
            needs_layout_passes=False, use_tc_tiling_on_sc=False),
        scratch_types=[
            pltpu.VMEM((CHUNK, HE), jnp.float32),
            pltpu.VMEM((CHUNK, HE), jnp.float32),
            pltpu.VMEM((CHUNK,), jnp.int32),
            pltpu.VMEM((CHUNK,), jnp.int32),
            pltpu.VMEM((EPW,), jnp.float32),
            pltpu.VMEM((EPW,), jnp.int32),
            pltpu.VMEM((EPW,), jnp.int32),
            pltpu.VMEM((EPW,), jnp.int32),
            pltpu.VMEM((N,), jnp.float32),
            pltpu.VMEM((N,), jnp.float32),
            pltpu.VMEM((16,), jnp.float32),
            pltpu.VMEM_SHARED((R * N, HE), jnp.float32),
            pltpu.VMEM_SHARED((N, HE), jnp.float32),
            pltpu.SemaphoreType.DMA,
            pltpu.SemaphoreType.DMA,
        ],
    )


def _sc_edge(*args):
    return _build_sc_edge()(*args)


def kernel(x, edge_index, edge_type, W_in, b_in, W_tfc, b_tfc, in_proj_w,
           in_proj_b, out_proj_w, out_proj_b, rgcn0_w, rgcn0_root, rgcn0_b,
           gat0_w, gat0_att_src, gat0_att_dst, gat0_b, rgcn1_w, rgcn1_root,
           rgcn1_b, gat1_w, gat1_att_src, gat1_att_dst, gat1_b, W_out, b_out):
    src = edge_index[0].astype(jnp.int32)
    dst = edge_index[1].astype(jnp.int32)
    et = edge_type.astype(jnp.int32)
    gidx = et * N + src
    seg = et * N + dst

    wq, wk, wv = (in_proj_w[i * H:(i + 1) * H].reshape(HEADS, HEAD_DIM, H)
                  for i in range(3))
    bq, bk, bv = (in_proj_b[i * H:(i + 1) * H].reshape(HEADS, 1, HEAD_DIM)
                  for i in range(3))
    q, k, v = _proj(x, W_in, b_in.reshape(1, H), W_tfc, b_tfc.reshape(1, H),
                    wq, wk, wv, bq, bk, bv)
    ow = out_proj_w.T.reshape(HEADS, HEAD_DIM, H)
    xh = _attn(q, k, v, ow, out_proj_b.reshape(1, H))

    hr, hx, hs, hd, c = _tables(xh, rgcn0_w, gat0_w,
                                gat0_att_src.reshape(H, 1),
                                gat0_att_dst.reshape(H, 1))
    rp, gp = _sc_edge(gidx, seg, src, dst, hr, hx,
                      hs.reshape(N), hd.reshape(N),
                      jnp.broadcast_to(c.reshape(1), (16,)))
    xh1, hr1, hx1, hs1, hd1, c1 = _combine_tables(
        xh, rp, gp, rgcn0_root, rgcn0_b.reshape(1, H), gat0_b.reshape(1, H),
        hx, hs, hd, c, rgcn1_w, gat1_w,
        gat1_att_src.reshape(H, 1), gat1_att_dst.reshape(H, 1))
    rp1, gp1 = _sc_edge(gidx, seg, src, dst, hr1, hx1,
                        hs1.reshape(N), hd1.reshape(N),
                        jnp.broadcast_to(c1.reshape(1), (16,)))
    out = _combine_out(xh1, rp1, gp1, rgcn1_root, rgcn1_b.reshape(1, H),
                       gat1_b.reshape(1, H), hx1, hs1, hd1, c1,
                       W_out, b_out.reshape(1, OUT_DIM))
    return out.reshape(1, N, OUT_DIM)


# speedup vs baseline: 1.7702x; 1.7702x over previous
import functools

import jax
import jax.numpy as jnp
from jax import lax
from jax.experimental import pallas as pl
from jax.experimental.pallas import tpu as pltpu
from jax.experimental.pallas import tpu_sc as plsc

N = 2048
E = 131072
IN_DIM = 28
H = 64
OUT_DIM = 28
R = 4
HEADS = 4
HEAD_DIM = H // HEADS

HE = H + 16
NC = 2
NS = 16
NW = NC * NS
EPW = E // NW
CHUNK = 128
NCHUNK = EPW // CHUNK

_dot = functools.partial(
    lax.dot_general, preferred_element_type=jnp.float32)


def _mm(a, b):
    return _dot(a, b, (((a.ndim - 1,), (0,)), ((), ())))


def _mm_t(a, b):
    return _dot(a, b, (((1,), (1,)), ((), ())))


def _leaky(x):
    return jnp.where(x >= 0, x, 0.2 * x)


def _proj_body(x_ref, w_in_ref, b_in_ref, w_tfc_ref, b_tfc_ref,
               wq_ref, wk_ref, wv_ref, bq_ref, bk_ref, bv_ref,
               q_ref, k_ref, v_ref):
    xh = _mm(x_ref[0], w_in_ref[...]) + b_in_ref[...]
    xh = _mm(xh, w_tfc_ref[...]) + b_tfc_ref[...]
    for h in range(HEADS):
        q_ref[h] = _mm_t(xh, wq_ref[h]) + bq_ref[h]
        k_ref[h] = _mm_t(xh, wk_ref[h]) + bk_ref[h]
        v_ref[h] = _mm_t(xh, wv_ref[h]) + bv_ref[h]


def _proj(x, w_in, b_in, w_tfc, b_tfc, wq, wk, wv, bq, bk, bv):
    hshape = jax.ShapeDtypeStruct((HEADS, N, HEAD_DIM), jnp.float32)
    return pl.pallas_call(
        _proj_body,
        out_shape=(hshape, hshape, hshape),
    )(x, w_in, b_in, w_tfc, b_tfc, wq, wk, wv, bq, bk, bv)


def _attn_body(q_ref, k_ref, v_ref, ow_ref, ob_ref, o_ref):
    h = pl.program_id(0)
    s = _mm_t(q_ref[0] * (1.0 / (HEAD_DIM ** 0.5)), k_ref[0])
    m = jnp.max(s, axis=1, keepdims=True)
    e = jnp.exp(s - m)
    p = e / jnp.sum(e, axis=1, keepdims=True)
    contrib = _mm(_mm(p, v_ref[0]), ow_ref[0])

    @pl.when(h == 0)
    def _():
        o_ref[...] = contrib + ob_ref[...]

    @pl.when(h != 0)
    def _():
        o_ref[...] = o_ref[...] + contrib


def _attn(q, k, v, ow, ob):
    spec = pl.BlockSpec((1, N, HEAD_DIM), lambda h: (h, 0, 0))
    return pl.pallas_call(
        _attn_body,
        grid=(HEADS,),
        in_specs=[spec, spec, spec,
                  pl.BlockSpec((1, HEAD_DIM, H), lambda h: (h, 0, 0)),
                  pl.BlockSpec((1, H), lambda h: (0, 0))],
        out_specs=pl.BlockSpec((N, H), lambda h: (0, 0)),
        out_shape=jax.ShapeDtypeStruct((N, H), jnp.float32),
    )(q, k, v, ow, ob)


_TABLES_OUT = (
    jax.ShapeDtypeStruct((R * N, HE), jnp.float32),
    jax.ShapeDtypeStruct((N, HE), jnp.float32),
    jax.ShapeDtypeStruct((N, 1), jnp.float32),
    jax.ShapeDtypeStruct((N, 1), jnp.float32),
    jax.ShapeDtypeStruct((1, 1), jnp.float32),
)


def _tables_core(xh, rw_ref, gw_ref, asrc_ref, adst_ref,
                 hr_ref, hx_ref, hs_ref, hd_ref, c_ref):
    onescol = jnp.concatenate(
        [jnp.ones((N, 1), jnp.float32), jnp.zeros((N, HE - H - 1), jnp.float32)],
        axis=1)
    for r in range(R):
        hr_ref[r * N:(r + 1) * N, 0:H] = _mm(xh, rw_ref[r])
        hr_ref[r * N:(r + 1) * N, H:HE] = onescol
    h = _mm(xh, gw_ref[...])
    hx_ref[:, 0:H] = h
    hx_ref[:, H:HE] = onescol
    hs = _mm(h, asrc_ref[...])
    hd = _mm(h, adst_ref[...])
    hs_ref[...] = hs
    hd_ref[...] = hd
    c = _leaky(jnp.max(hs) + jnp.max(hd))
    c_ref[...] = jnp.full((1, 1), c, jnp.float32)


def _combine_core(xh, rp_ref, gp_ref, root_ref, rb_ref, gb_ref,
                  hx_ref, hs_ref, hd_ref, c_ref):
    agg = jnp.zeros((N, H), jnp.float32)
    for r in range(R):
        blk = rp_ref[0, r * N:(r + 1) * N, :] + rp_ref[1, r * N:(r + 1) * N, :]
        cnt = jnp.maximum(blk[:, H:H + 1], 1.0)
        agg = agg + blk[:, 0:H] / cnt
    xr = agg + _mm(xh, root_ref[...]) + rb_ref[...]

    gp = gp_ref[0] + gp_ref[1]
    c = c_ref[0, 0]
    exn = jnp.exp(_leaky(hs_ref[...] + hd_ref[...]) - c)
    num = gp[:, 0:H] + exn * hx_ref[:, 0:H]
    den = gp[:, H:H + 1] + exn
    xg = num / den + gb_ref[...]
    return jnp.maximum(xr + xg, 0.0)


def _tables_body(xh_ref, rw_ref, gw_ref, asrc_ref, adst_ref,
                 hr_ref, hx_ref, hs_ref, hd_ref, c_ref):
    _tables_core(xh_ref[...], rw_ref, gw_ref, asrc_ref, adst_ref,
                 hr_ref, hx_ref, hs_ref, hd_ref, c_ref)


def _tables(xh, rw, gw, asrc, adst):
    return pl.pallas_call(
        _tables_body,
        out_shape=_TABLES_OUT,
    )(xh, rw, gw, asrc, adst)


def _combine_tables_body(xh_ref, rp_ref, gp_ref, root_ref, rb_ref, gb_ref,
                         hx_ref, hs_ref, hd_ref, c_ref,
                         rw2_ref, gw2_ref, asrc2_ref, adst2_ref,
                         xh2_ref, hr2_ref, hx2_ref, hs2_ref, hd2_ref, c2_ref):
    xh2 = _combine_core(xh_ref[...], rp_ref, gp_ref, root_ref, rb_ref, gb_ref,
                        hx_ref, hs_ref, hd_ref, c_ref)
    xh2_ref[...] = xh2
    _tables_core(xh2, rw2_ref, gw2_ref, asrc2_ref, adst2_ref,
                 hr2_ref, hx2_ref, hs2_ref, hd2_ref, c2_ref)


def _combine_tables(xh, rp, gp, root, rb, gb, hx, hs, hd, c,
                    rw2, gw2, asrc2, adst2):
    return pl.pallas_call(
        _combine_tables_body,
        out_shape=(jax.ShapeDtypeStruct((N, H), jnp.float32),) + _TABLES_OUT,
    )(xh, rp, gp, root, rb, gb, hx, hs, hd, c, rw2, gw2, asrc2, adst2)


def _combine_out_body(xh_ref, rp_ref, gp_ref, root_ref, rb_ref, gb_ref,
                      hx_ref, hs_ref, hd_ref, c_ref, wout_ref, bout_ref,
                      out_ref):
    xh2 = _combine_core(xh_ref[...], rp_ref, gp_ref, root_ref, rb_ref, gb_ref,
                        hx_ref, hs_ref, hd_ref, c_ref)
    out_ref[...] = _mm(xh2, wout_ref[...]) + bout_ref[...]


def _combine_out(xh, rp, gp, root, rb, gb, hx, hs, hd, c, wout, bout):
    return pl.pallas_call(
        _combine_out_body,
        out_shape=jax.ShapeDtypeStruct((N, OUT_DIM), jnp.float32),
    )(xh, rp, gp, root, rb, gb, hx, hs, hd, c, wout, bout)


def _sc_edge_body(gidx_hbm, seg_hbm, src_hbm, dst_hbm, hr_hbm, hx_hbm,
                  hs_hbm, hd_hbm, c_hbm,
                  rgcn_out, gat_out,
                  rows0_v, rows1_v, sg0_v, sg1_v, ex_v, ia_v, ib_v,
                  ga_v, hs_v, hd_v, c_v,
                  racc, gacc, sem0, sem1, ssem0, ssem1):
    cid = lax.axis_index("c")
    sid = lax.axis_index("s")
    wid = cid * NS + sid
    base = wid * EPW
    rows = (rows0_v, rows1_v)
    sgs = (sg0_v, sg1_v)
    sems = (sem0, sem1)
    ssems = (ssem0, ssem1)

    pltpu.sync_copy(src_hbm.at[pl.ds(base, EPW)], ia_v)
    pltpu.sync_copy(dst_hbm.at[pl.ds(base, EPW)], ib_v)
    pltpu.sync_copy(gidx_hbm.at[pl.ds(base, EPW)], ga_v)
    pltpu.sync_copy(hs_hbm, hs_v)
    pltpu.sync_copy(hd_hbm, hd_v)
    pltpu.sync_copy(c_hbm, c_v)

    def zrow(i, _):
        r = i // (HE // 16)
        k = i % (HE // 16)
        rows0_v[r, pl.ds(k * 16, 16)] = jnp.zeros((16,), jnp.float32)
        return 0
    lax.fori_loop(0, CHUNK * (HE // 16), zrow, 0)

    rrows = (R * N) // NS
    grows = N // NS
    for j in range(rrows // CHUNK):
        pltpu.sync_copy(rows0_v, racc.at[pl.ds(sid * rrows + j * CHUNK, CHUNK)])
    pltpu.sync_copy(rows0_v, gacc.at[pl.ds(sid * grows, grows)])
    plsc.subcore_barrier()

    c = c_v[...][0]

    def score(g, _):
        s16 = ia_v[pl.ds(g * 16, 16)]
        d16 = ib_v[pl.ds(g * 16, 16)]
        sc = plsc.load_gather(hs_v, [s16]) + plsc.load_gather(hd_v, [d16])
        ex_v[pl.ds(g * 16, 16)] = jnp.exp(_leaky(sc) - c)
        return 0
    lax.fori_loop(0, EPW // 16, score, 0)

    TOT = 2 * NCHUNK

    def issue(c, b):
        @pl.when(c < NCHUNK)
        def _():
            pltpu.sync_copy(dst_hbm.at[pl.ds(base + c * CHUNK, CHUNK)], sgs[b])
            pltpu.async_copy(hx_hbm.at[ia_v.at[pl.ds(c * CHUNK, CHUNK)]],
                             rows[b], sems[b])

        @pl.when(c >= NCHUNK)
        def _():
            cr = c - NCHUNK
            pltpu.sync_copy(seg_hbm.at[pl.ds(base + cr * CHUNK, CHUNK)], sgs[b])
            pltpu.async_copy(hr_hbm.at[ga_v.at[pl.ds(cr * CHUNK, CHUNK)]],
                             rows[b], sems[b])

    def wait_gather(b):
        pltpu.make_async_copy(hx_hbm.at[ia_v.at[pl.ds(0, CHUNK)]],
                              rows[b], sems[b]).wait()

    def wait_scatter(b):
        pltpu.make_async_copy(hx_hbm.at[pl.ds(0, CHUNK)], rows[b],
                              ssems[b]).wait()

    def drain_process(c, b):
        wait_gather(b)

        @pl.when(c < NCHUNK)
        def _():
            def scalegrp(g, _):
                ex16 = ex_v[pl.ds(c * CHUNK + g * 16, 16)]
                for i in range(16):
                    a = ex16[i]
                    e = g * 16 + i
                    for j in range(HE // 16):
                        sl = pl.ds(j * 16, 16)
                        rows[b][e, sl] = rows[b][e, sl] * a
                return 0
            lax.fori_loop(0, CHUNK // 16, scalegrp, 0)
            pltpu.async_copy(rows[b], gacc.at[sgs[b]], ssems[b], add=True)

        @pl.when(c >= NCHUNK)
        def _():
            pltpu.async_copy(rows[b], racc.at[sgs[b]], ssems[b], add=True)

    pltpu.sync_copy(dst_hbm.at[pl.ds(base, CHUNK)], sg0_v)
    pltpu.async_copy(hx_hbm.at[ia_v.at[pl.ds(0, CHUNK)]], rows0_v, sem0)

    def pipe(i, _):
        for b in range(2):
            c = 2 * i + b

            @pl.when(c + 1 < TOT)
            def _():
                @pl.when(c >= 1)
                def _():
                    wait_scatter(1 - b)
                issue(c + 1, 1 - b)
            drain_process(c, b)
        return 0
    lax.fori_loop(0, TOT // 2, pipe, 0)
    wait_scatter(0)
    wait_scatter(1)

    plsc.subcore_barrier()
    for j in range(rrows // CHUNK):
        off = sid * rrows + j * CHUNK
        pltpu.sync_copy(racc.at[pl.ds(off, CHUNK)],
                        rgcn_out.at[cid, pl.ds(off, CHUNK)])
    pltpu.sync_copy(gacc.at[pl.ds(sid * grows, grows)],
                    gat_out.at[cid, pl.ds(sid * grows, grows)])


@functools.lru_cache(maxsize=1)
def _build_sc_edge():
    return pl.kernel(
        _sc_edge_body,
        out_type=(
            jax.ShapeDtypeStruct((NC, R * N, HE), jnp.float32),
            jax.ShapeDtypeStruct((NC, N, HE), jnp.float32),
        ),
        mesh=plsc.VectorSubcoreMesh(core_axis_name="c", subcore_axis_name="s",
                                    num_cores=NC, num_subcores=NS),
        compiler_params=pltpu.CompilerParams(
            needs_layout_passes=False, use_tc_tiling_on_sc=False),
        scratch_types=[
            pltpu.VMEM((CHUNK, HE), jnp.float32),
            pltpu.VMEM((CHUNK, HE), jnp.float32),
            pltpu.VMEM((CHUNK,), jnp.int32),
            pltpu.VMEM((CHUNK,), jnp.int32),
            pltpu.VMEM((EPW,), jnp.float32),
            pltpu.VMEM((EPW,), jnp.int32),
            pltpu.VMEM((EPW,), jnp.int32),
            pltpu.VMEM((EPW,), jnp.int32),
            pltpu.VMEM((N,), jnp.float32),
            pltpu.VMEM((N,), jnp.float32),
            pltpu.VMEM((16,), jnp.float32),
            pltpu.VMEM_SHARED((R * N, HE), jnp.float32),
            pltpu.VMEM_SHARED((N, HE), jnp.float32),
            pltpu.SemaphoreType.DMA,
            pltpu.SemaphoreType.DMA,
            pltpu.SemaphoreType.DMA,
            pltpu.SemaphoreType.DMA,
        ],
    )


def _sc_edge(*args):
    return _build_sc_edge()(*args)


def kernel(x, edge_index, edge_type, W_in, b_in, W_tfc, b_tfc, in_proj_w,
           in_proj_b, out_proj_w, out_proj_b, rgcn0_w, rgcn0_root, rgcn0_b,
           gat0_w, gat0_att_src, gat0_att_dst, gat0_b, rgcn1_w, rgcn1_root,
           rgcn1_b, gat1_w, gat1_att_src, gat1_att_dst, gat1_b, W_out, b_out):
    src = edge_index[0].astype(jnp.int32)
    dst = edge_index[1].astype(jnp.int32)
    et = edge_type.astype(jnp.int32)
    gidx = et * N + src
    seg = et * N + dst

    wq, wk, wv = (in_proj_w[i * H:(i + 1) * H].reshape(HEADS, HEAD_DIM, H)
                  for i in range(3))
    bq, bk, bv = (in_proj_b[i * H:(i + 1) * H].reshape(HEADS, 1, HEAD_DIM)
                  for i in range(3))
    q, k, v = _proj(x, W_in, b_in.reshape(1, H), W_tfc, b_tfc.reshape(1, H),
                    wq, wk, wv, bq, bk, bv)
    ow = out_proj_w.T.reshape(HEADS, HEAD_DIM, H)
    xh = _attn(q, k, v, ow, out_proj_b.reshape(1, H))

    hr, hx, hs, hd, c = _tables(xh, rgcn0_w, gat0_w,
                                gat0_att_src.reshape(H, 1),
                                gat0_att_dst.reshape(H, 1))
    rp, gp = _sc_edge(gidx, seg, src, dst, hr, hx,
                      hs.reshape(N), hd.reshape(N),
                      jnp.broadcast_to(c.reshape(1), (16,)))
    xh1, hr1, hx1, hs1, hd1, c1 = _combine_tables(
        xh, rp, gp, rgcn0_root, rgcn0_b.reshape(1, H), gat0_b.reshape(1, H),
        hx, hs, hd, c, rgcn1_w, gat1_w,
        gat1_att_src.reshape(H, 1), gat1_att_dst.reshape(H, 1))
    rp1, gp1 = _sc_edge(gidx, seg, src, dst, hr1, hx1,
                        hs1.reshape(N), hd1.reshape(N),
                        jnp.broadcast_to(c1.reshape(1), (16,)))
    out = _combine_out(xh1, rp1, gp1, rgcn1_root, rgcn1_b.reshape(1, H),
                       gat1_b.reshape(1, H), hx1, hs1, hd1, c1,
                       W_out, b_out.reshape(1, OUT_DIM))
    return out.reshape(1, N, OUT_DIM)


# speedup vs baseline: 1.9393x; 1.0955x over previous
import functools

import jax
import jax.numpy as jnp
from jax import lax
from jax.experimental import pallas as pl
from jax.experimental.pallas import tpu as pltpu
from jax.experimental.pallas import tpu_sc as plsc

N = 2048
E = 131072
IN_DIM = 28
H = 64
OUT_DIM = 28
R = 4
HEADS = 4
HEAD_DIM = H // HEADS

HE = H + 16
NC = 2
NS = 16
NW = NC * NS
EPW = E // NW
CHUNK = 128
NCHUNK = EPW // CHUNK

_dot = functools.partial(
    lax.dot_general, preferred_element_type=jnp.float32)


def _mm(a, b):
    return _dot(a, b, (((a.ndim - 1,), (0,)), ((), ())))


def _mm_t(a, b):
    return _dot(a, b, (((1,), (1,)), ((), ())))


def _leaky(x):
    return jnp.where(x >= 0, x, 0.2 * x)


def _proj_body(x_ref, w_in_ref, b_in_ref, w_tfc_ref, b_tfc_ref,
               wq_ref, wk_ref, wv_ref, bq_ref, bk_ref, bv_ref,
               q_ref, k_ref, v_ref):
    xh = _mm(x_ref[0], w_in_ref[...]) + b_in_ref[...]
    xh = _mm(xh, w_tfc_ref[...]) + b_tfc_ref[...]
    for h in range(HEADS):
        q_ref[h] = _mm_t(xh, wq_ref[h]) + bq_ref[h]
        k_ref[h] = _mm_t(xh, wk_ref[h]) + bk_ref[h]
        v_ref[h] = _mm_t(xh, wv_ref[h]) + bv_ref[h]


def _proj(x, w_in, b_in, w_tfc, b_tfc, wq, wk, wv, bq, bk, bv):
    hshape = jax.ShapeDtypeStruct((HEADS, N, HEAD_DIM), jnp.float32)
    return pl.pallas_call(
        _proj_body,
        out_shape=(hshape, hshape, hshape),
    )(x, w_in, b_in, w_tfc, b_tfc, wq, wk, wv, bq, bk, bv)


def _attn_body(q_ref, k_ref, v_ref, ow_ref, ob_ref, o_ref):
    h = pl.program_id(0)
    s = _mm_t(q_ref[0] * (1.0 / (HEAD_DIM ** 0.5)), k_ref[0])
    m = jnp.max(s, axis=1, keepdims=True)
    e = jnp.exp(s - m)
    p = e / jnp.sum(e, axis=1, keepdims=True)
    contrib = _mm(_mm(p, v_ref[0]), ow_ref[0])

    @pl.when(h == 0)
    def _():
        o_ref[...] = contrib + ob_ref[...]

    @pl.when(h != 0)
    def _():
        o_ref[...] = o_ref[...] + contrib


def _attn(q, k, v, ow, ob):
    spec = pl.BlockSpec((1, N, HEAD_DIM), lambda h: (h, 0, 0))
    return pl.pallas_call(
        _attn_body,
        grid=(HEADS,),
        in_specs=[spec, spec, spec,
                  pl.BlockSpec((1, HEAD_DIM, H), lambda h: (h, 0, 0)),
                  pl.BlockSpec((1, H), lambda h: (0, 0))],
        out_specs=pl.BlockSpec((N, H), lambda h: (0, 0)),
        out_shape=jax.ShapeDtypeStruct((N, H), jnp.float32),
    )(q, k, v, ow, ob)


_TABLES_OUT = (
    jax.ShapeDtypeStruct((R * N, HE), jnp.float32),
    jax.ShapeDtypeStruct((N, HE), jnp.float32),
    jax.ShapeDtypeStruct((N, 1), jnp.float32),
    jax.ShapeDtypeStruct((N, 1), jnp.float32),
    jax.ShapeDtypeStruct((1, 1), jnp.float32),
)


def _tables_core(xh, rw_ref, gw_ref, asrc_ref, adst_ref,
                 hr_ref, hx_ref, hs_ref, hd_ref, c_ref):
    onescol = jnp.concatenate(
        [jnp.ones((N, 1), jnp.float32), jnp.zeros((N, HE - H - 1), jnp.float32)],
        axis=1)
    for r in range(R):
        hr_ref[r * N:(r + 1) * N, 0:H] = _mm(xh, rw_ref[r])
        hr_ref[r * N:(r + 1) * N, H:HE] = onescol
    h = _mm(xh, gw_ref[...])
    hx_ref[:, 0:H] = h
    hx_ref[:, H:HE] = onescol
    hs = _mm(h, asrc_ref[...])
    hd = _mm(h, adst_ref[...])
    hs_ref[...] = hs
    hd_ref[...] = hd
    c = _leaky(jnp.max(hs) + jnp.max(hd))
    c_ref[...] = jnp.full((1, 1), c, jnp.float32)


def _combine_core(xh, rp_ref, gp_ref, root_ref, rb_ref, gb_ref,
                  hx_ref, hs_ref, hd_ref, c_ref):
    agg = jnp.zeros((N, H), jnp.float32)
    for r in range(R):
        blk = rp_ref[0, r * N:(r + 1) * N, :] + rp_ref[1, r * N:(r + 1) * N, :]
        cnt = jnp.maximum(blk[:, H:H + 1], 1.0)
        agg = agg + blk[:, 0:H] / cnt
    xr = agg + _mm(xh, root_ref[...]) + rb_ref[...]

    gp = gp_ref[0] + gp_ref[1]
    c = c_ref[0, 0]
    exn = jnp.exp(_leaky(hs_ref[...] + hd_ref[...]) - c)
    num = gp[:, 0:H] + exn * hx_ref[:, 0:H]
    den = gp[:, H:H + 1] + exn
    xg = num / den + gb_ref[...]
    return jnp.maximum(xr + xg, 0.0)


def _tables_body(xh_ref, rw_ref, gw_ref, asrc_ref, adst_ref,
                 hr_ref, hx_ref, hs_ref, hd_ref, c_ref):
    _tables_core(xh_ref[...], rw_ref, gw_ref, asrc_ref, adst_ref,
                 hr_ref, hx_ref, hs_ref, hd_ref, c_ref)


def _tables(xh, rw, gw, asrc, adst):
    return pl.pallas_call(
        _tables_body,
        out_shape=_TABLES_OUT,
    )(xh, rw, gw, asrc, adst)


def _combine_tables_body(xh_ref, rp_ref, gp_ref, root_ref, rb_ref, gb_ref,
                         hx_ref, hs_ref, hd_ref, c_ref,
                         rw2_ref, gw2_ref, asrc2_ref, adst2_ref,
                         xh2_ref, hr2_ref, hx2_ref, hs2_ref, hd2_ref, c2_ref):
    xh2 = _combine_core(xh_ref[...], rp_ref, gp_ref, root_ref, rb_ref, gb_ref,
                        hx_ref, hs_ref, hd_ref, c_ref)
    xh2_ref[...] = xh2
    _tables_core(xh2, rw2_ref, gw2_ref, asrc2_ref, adst2_ref,
                 hr2_ref, hx2_ref, hs2_ref, hd2_ref, c2_ref)


def _combine_tables(xh, rp, gp, root, rb, gb, hx, hs, hd, c,
                    rw2, gw2, asrc2, adst2):
    return pl.pallas_call(
        _combine_tables_body,
        out_shape=(jax.ShapeDtypeStruct((N, H), jnp.float32),) + _TABLES_OUT,
    )(xh, rp, gp, root, rb, gb, hx, hs, hd, c, rw2, gw2, asrc2, adst2)


def _combine_out_body(xh_ref, rp_ref, gp_ref, root_ref, rb_ref, gb_ref,
                      hx_ref, hs_ref, hd_ref, c_ref, wout_ref, bout_ref,
                      out_ref):
    xh2 = _combine_core(xh_ref[...], rp_ref, gp_ref, root_ref, rb_ref, gb_ref,
                        hx_ref, hs_ref, hd_ref, c_ref)
    out_ref[...] = _mm(xh2, wout_ref[...]) + bout_ref[...]


def _combine_out(xh, rp, gp, root, rb, gb, hx, hs, hd, c, wout, bout):
    return pl.pallas_call(
        _combine_out_body,
        out_shape=jax.ShapeDtypeStruct((N, OUT_DIM), jnp.float32),
    )(xh, rp, gp, root, rb, gb, hx, hs, hd, c, wout, bout)


def _sc_edge_body(gidx_hbm, seg_hbm, src_hbm, dst_hbm, hr_hbm, hx_hbm,
                  hs_hbm, hd_hbm, c_hbm,
                  rgcn_out, gat_out,
                  rows0_v, rows1_v, ex_v, ia_v, ib_v, ga_v, sb_v,
                  hs_v, hd_v, c_v,
                  racc, gacc, sem0, sem1, ssem0, ssem1):
    cid = lax.axis_index("c")
    sid = lax.axis_index("s")
    wid = cid * NS + sid
    rows = (rows0_v, rows1_v)
    sems = (sem0, sem1)
    ssems = (ssem0, ssem1)

    pltpu.sync_copy(src_hbm.at[wid, :, :], ia_v)
    pltpu.sync_copy(dst_hbm.at[wid, :, :], ib_v)
    pltpu.sync_copy(gidx_hbm.at[wid, :, :], ga_v)
    pltpu.sync_copy(seg_hbm.at[wid, :, :], sb_v)
    pltpu.sync_copy(hs_hbm, hs_v)
    pltpu.sync_copy(hd_hbm, hd_v)
    pltpu.sync_copy(c_hbm, c_v)

    def zrow(i, _):
        r = i // (HE // 16)
        k = i % (HE // 16)
        rows0_v[r, pl.ds(k * 16, 16)] = jnp.zeros((16,), jnp.float32)
        return 0
    lax.fori_loop(0, CHUNK * (HE // 16), zrow, 0)

    rrows = (R * N) // NS
    grows = N // NS
    for j in range(rrows // CHUNK):
        pltpu.sync_copy(rows0_v, racc.at[pl.ds(sid * rrows + j * CHUNK, CHUNK)])
    pltpu.sync_copy(rows0_v, gacc.at[pl.ds(sid * grows, grows)])
    plsc.subcore_barrier()

    c = c_v[...][0]

    GPC = CHUNK // 16

    def score(g, _):
        gc = g // GPC
        go = g % GPC
        s16 = ia_v[gc, pl.ds(go * 16, 16)]
        d16 = ib_v[gc, pl.ds(go * 16, 16)]
        sc = plsc.load_gather(hs_v, [s16]) + plsc.load_gather(hd_v, [d16])
        ex_v[pl.ds(g * 16, 16)] = jnp.exp(_leaky(sc) - c)
        return 0
    lax.fori_loop(0, EPW // 16, score, 0)

    TOT = 2 * NCHUNK

    def issue(c, b):
        @pl.when(c < NCHUNK)
        def _():
            pltpu.async_copy(hx_hbm.at[ia_v.at[c]], rows[b], sems[b])

        @pl.when(c >= NCHUNK)
        def _():
            pltpu.async_copy(hr_hbm.at[ga_v.at[c - NCHUNK]], rows[b], sems[b])

    def wait_gather(b):
        pltpu.make_async_copy(hx_hbm.at[ia_v.at[0]], rows[b], sems[b]).wait()

    def wait_scatter(b):
        pltpu.make_async_copy(hx_hbm.at[pl.ds(0, CHUNK)], rows[b],
                              ssems[b]).wait()

    def drain_process(c, b):
        wait_gather(b)

        @pl.when(c < NCHUNK)
        def _():
            def scalegrp(g, _):
                ex16 = ex_v[pl.ds(c * CHUNK + g * 16, 16)]
                for i in range(16):
                    a = ex16[i]
                    e = g * 16 + i
                    for j in range(HE // 16):
                        sl = pl.ds(j * 16, 16)
                        rows[b][e, sl] = rows[b][e, sl] * a
                return 0
            lax.fori_loop(0, CHUNK // 16, scalegrp, 0)
            pltpu.async_copy(rows[b], gacc.at[ib_v.at[c]], ssems[b], add=True)

        @pl.when(c >= NCHUNK)
        def _():
            pltpu.async_copy(rows[b], racc.at[sb_v.at[c - NCHUNK]],
                             ssems[b], add=True)

    pltpu.async_copy(hx_hbm.at[ia_v.at[0]], rows0_v, sem0)

    def pipe(i, _):
        for b in range(2):
            c = 2 * i + b

            @pl.when(c + 1 < TOT)
            def _():
                @pl.when(c >= 1)
                def _():
                    wait_scatter(1 - b)
                issue(c + 1, 1 - b)
            drain_process(c, b)
        return 0
    lax.fori_loop(0, TOT // 2, pipe, 0)
    wait_scatter(0)
    wait_scatter(1)

    plsc.subcore_barrier()
    for j in range(rrows // CHUNK):
        off = sid * rrows + j * CHUNK
        pltpu.sync_copy(racc.at[pl.ds(off, CHUNK)],
                        rgcn_out.at[cid, pl.ds(off, CHUNK)])
    pltpu.sync_copy(gacc.at[pl.ds(sid * grows, grows)],
                    gat_out.at[cid, pl.ds(sid * grows, grows)])


@functools.lru_cache(maxsize=1)
def _build_sc_edge():
    return pl.kernel(
        _sc_edge_body,
        out_type=(
            jax.ShapeDtypeStruct((NC, R * N, HE), jnp.float32),
            jax.ShapeDtypeStruct((NC, N, HE), jnp.float32),
        ),
        mesh=plsc.VectorSubcoreMesh(core_axis_name="c", subcore_axis_name="s",
                                    num_cores=NC, num_subcores=NS),
        compiler_params=pltpu.CompilerParams(
            needs_layout_passes=False, use_tc_tiling_on_sc=False),
        scratch_types=[
            pltpu.VMEM((CHUNK, HE), jnp.float32),
            pltpu.VMEM((CHUNK, HE), jnp.float32),
            pltpu.VMEM((EPW,), jnp.float32),
            pltpu.VMEM((NCHUNK, CHUNK), jnp.int32),
            pltpu.VMEM((NCHUNK, CHUNK), jnp.int32),
            pltpu.VMEM((NCHUNK, CHUNK), jnp.int32),
            pltpu.VMEM((NCHUNK, CHUNK), jnp.int32),
            pltpu.VMEM((N,), jnp.float32),
            pltpu.VMEM((N,), jnp.float32),
            pltpu.VMEM((16,), jnp.float32),
            pltpu.VMEM_SHARED((R * N, HE), jnp.float32),
            pltpu.VMEM_SHARED((N, HE), jnp.float32),
            pltpu.SemaphoreType.DMA,
            pltpu.SemaphoreType.DMA,
            pltpu.SemaphoreType.DMA,
            pltpu.SemaphoreType.DMA,
        ],
    )


def _sc_edge(*args):
    return _build_sc_edge()(*args)


def kernel(x, edge_index, edge_type, W_in, b_in, W_tfc, b_tfc, in_proj_w,
           in_proj_b, out_proj_w, out_proj_b, rgcn0_w, rgcn0_root, rgcn0_b,
           gat0_w, gat0_att_src, gat0_att_dst, gat0_b, rgcn1_w, rgcn1_root,
           rgcn1_b, gat1_w, gat1_att_src, gat1_att_dst, gat1_b, W_out, b_out):
    src = edge_index[0].astype(jnp.int32)
    dst = edge_index[1].astype(jnp.int32)
    et = edge_type.astype(jnp.int32)
    shp = (NW, NCHUNK, CHUNK)
    gidx = (et * N + src).reshape(shp)
    seg = (et * N + dst).reshape(shp)
    src = src.reshape(shp)
    dst = dst.reshape(shp)

    wq, wk, wv = (in_proj_w[i * H:(i + 1) * H].reshape(HEADS, HEAD_DIM, H)
                  for i in range(3))
    bq, bk, bv = (in_proj_b[i * H:(i + 1) * H].reshape(HEADS, 1, HEAD_DIM)
                  for i in range(3))
    q, k, v = _proj(x, W_in, b_in.reshape(1, H), W_tfc, b_tfc.reshape(1, H),
                    wq, wk, wv, bq, bk, bv)
    ow = out_proj_w.T.reshape(HEADS, HEAD_DIM, H)
    xh = _attn(q, k, v, ow, out_proj_b.reshape(1, H))

    hr, hx, hs, hd, c = _tables(xh, rgcn0_w, gat0_w,
                                gat0_att_src.reshape(H, 1),
                                gat0_att_dst.reshape(H, 1))
    rp, gp = _sc_edge(gidx, seg, src, dst, hr, hx,
                      hs.reshape(N), hd.reshape(N),
                      jnp.broadcast_to(c.reshape(1), (16,)))
    xh1, hr1, hx1, hs1, hd1, c1 = _combine_tables(
        xh, rp, gp, rgcn0_root, rgcn0_b.reshape(1, H), gat0_b.reshape(1, H),
        hx, hs, hd, c, rgcn1_w, gat1_w,
        gat1_att_src.reshape(H, 1), gat1_att_dst.reshape(H, 1))
    rp1, gp1 = _sc_edge(gidx, seg, src, dst, hr1, hx1,
                        hs1.reshape(N), hd1.reshape(N),
                        jnp.broadcast_to(c1.reshape(1), (16,)))
    out = _combine_out(xh1, rp1, gp1, rgcn1_root, rgcn1_b.reshape(1, H),
                       gat1_b.reshape(1, H), hx1, hs1, hd1, c1,
                       W_out, b_out.reshape(1, OUT_DIM))
    return out.reshape(1, N, OUT_DIM)


# speedup vs baseline: 1.9982x; 1.0304x over previous
import functools

import jax
import jax.numpy as jnp
from jax import lax
from jax.experimental import pallas as pl
from jax.experimental.pallas import tpu as pltpu
from jax.experimental.pallas import tpu_sc as plsc

N = 2048
E = 131072
IN_DIM = 28
H = 64
OUT_DIM = 28
R = 4
HEADS = 4
HEAD_DIM = H // HEADS

HE = H + 16
NC = 2
NS = 16
NW = NC * NS
EPW = E // NW
CHUNK = 128
NCHUNK = EPW // CHUNK

_dot = functools.partial(
    lax.dot_general, preferred_element_type=jnp.float32)


def _mm(a, b):
    return _dot(a, b, (((a.ndim - 1,), (0,)), ((), ())))


def _mm_t(a, b):
    return _dot(a, b, (((1,), (1,)), ((), ())))


def _leaky(x):
    return jnp.where(x >= 0, x, 0.2 * x)


def _prologue_body(x_ref, w_in_ref, b_in_ref, w_tfc_ref, b_tfc_ref,
                   wq_ref, wk_ref, wv_ref, bq_ref, bk_ref, bv_ref,
                   ow_ref, ob_ref, rw_ref, gw_ref, asrc_ref, adst_ref,
                   o_ref, hr_ref, hx_ref, hs_ref, hd_ref, c_ref,
                   qs, ks, vs):
    h = pl.program_id(0)

    @pl.when(h == 0)
    def _():
        xh = _mm(x_ref[0], w_in_ref[...]) + b_in_ref[...]
        xh = _mm(xh, w_tfc_ref[...]) + b_tfc_ref[...]
        for hh in range(HEADS):
            qs[hh] = _mm_t(xh, wq_ref[hh]) + bq_ref[hh]
            ks[hh] = _mm_t(xh, wk_ref[hh]) + bk_ref[hh]
            vs[hh] = _mm_t(xh, wv_ref[hh]) + bv_ref[hh]

    s = _mm_t(qs[h] * (1.0 / (HEAD_DIM ** 0.5)), ks[h])
    m = jnp.max(s, axis=1, keepdims=True)
    e = jnp.exp(s - m)
    p = e / jnp.sum(e, axis=1, keepdims=True)
    contrib = _mm(_mm(p, vs[h]), ow_ref[h])

    @pl.when(h == 0)
    def _():
        o_ref[...] = contrib + ob_ref[...]

    @pl.when(h != 0)
    def _():
        o_ref[...] = o_ref[...] + contrib

    @pl.when(h == HEADS - 1)
    def _():
        _tables_core(o_ref[...], rw_ref, gw_ref, asrc_ref, adst_ref,
                     hr_ref, hx_ref, hs_ref, hd_ref, c_ref)


def _prologue(x, w_in, b_in, w_tfc, b_tfc, wq, wk, wv, bq, bk, bv, ow, ob,
              rw, gw, asrc, adst):
    def full(a):
        nd = len(a.shape)
        return pl.BlockSpec(a.shape, lambda h, _n=nd: (0,) * _n)
    args = (x, w_in, b_in, w_tfc, b_tfc, wq, wk, wv, bq, bk, bv, ow, ob,
            rw, gw, asrc, adst)
    outs = ((N, H), (R * N, HE), (N, HE), (N, 1), (N, 1), (1, 1))
    hshape = pltpu.VMEM((HEADS, N, HEAD_DIM), jnp.float32)
    return pl.pallas_call(
        _prologue_body,
        grid=(HEADS,),
        in_specs=[full(a) for a in args],
        out_specs=tuple(pl.BlockSpec(s, lambda h, _n=len(s): (0,) * _n)
                        for s in outs),
        out_shape=tuple(jax.ShapeDtypeStruct(s, jnp.float32) for s in outs),
        scratch_shapes=[hshape, hshape, hshape],
    )(*args)


_TABLES_OUT = (
    jax.ShapeDtypeStruct((R * N, HE), jnp.float32),
    jax.ShapeDtypeStruct((N, HE), jnp.float32),
    jax.ShapeDtypeStruct((N, 1), jnp.float32),
    jax.ShapeDtypeStruct((N, 1), jnp.float32),
    jax.ShapeDtypeStruct((1, 1), jnp.float32),
)


def _tables_core(xh, rw_ref, gw_ref, asrc_ref, adst_ref,
                 hr_ref, hx_ref, hs_ref, hd_ref, c_ref):
    onescol = jnp.concatenate(
        [jnp.ones((N, 1), jnp.float32), jnp.zeros((N, HE - H - 1), jnp.float32)],
        axis=1)
    for r in range(R):
        hr_ref[r * N:(r + 1) * N, 0:H] = _mm(xh, rw_ref[r])
        hr_ref[r * N:(r + 1) * N, H:HE] = onescol
    h = _mm(xh, gw_ref[...])
    hx_ref[:, 0:H] = h
    hx_ref[:, H:HE] = onescol
    hs = _mm(h, asrc_ref[...])
    hd = _mm(h, adst_ref[...])
    hs_ref[...] = hs
    hd_ref[...] = hd
    c = _leaky(jnp.max(hs) + jnp.max(hd))
    c_ref[...] = jnp.full((1, 1), c, jnp.float32)


def _combine_core(xh, rp_ref, gp_ref, root_ref, rb_ref, gb_ref,
                  hx_ref, hs_ref, hd_ref, c_ref):
    agg = jnp.zeros((N, H), jnp.float32)
    for r in range(R):
        blk = rp_ref[0, r * N:(r + 1) * N, :] + rp_ref[1, r * N:(r + 1) * N, :]
        cnt = jnp.maximum(blk[:, H:H + 1], 1.0)
        agg = agg + blk[:, 0:H] / cnt
    xr = agg + _mm(xh, root_ref[...]) + rb_ref[...]

    gp = gp_ref[0] + gp_ref[1]
    c = c_ref[0, 0]
    exn = jnp.exp(_leaky(hs_ref[...] + hd_ref[...]) - c)
    num = gp[:, 0:H] + exn * hx_ref[:, 0:H]
    den = gp[:, H:H + 1] + exn
    xg = num / den + gb_ref[...]
    return jnp.maximum(xr + xg, 0.0)


def _tables_body(xh_ref, rw_ref, gw_ref, asrc_ref, adst_ref,
                 hr_ref, hx_ref, hs_ref, hd_ref, c_ref):
    _tables_core(xh_ref[...], rw_ref, gw_ref, asrc_ref, adst_ref,
                 hr_ref, hx_ref, hs_ref, hd_ref, c_ref)


def _tables(xh, rw, gw, asrc, adst):
    return pl.pallas_call(
        _tables_body,
        out_shape=_TABLES_OUT,
    )(xh, rw, gw, asrc, adst)


def _combine_tables_body(xh_ref, rp_ref, gp_ref, root_ref, rb_ref, gb_ref,
                         hx_ref, hs_ref, hd_ref, c_ref,
                         rw2_ref, gw2_ref, asrc2_ref, adst2_ref,
                         xh2_ref, hr2_ref, hx2_ref, hs2_ref, hd2_ref, c2_ref):
    xh2 = _combine_core(xh_ref[...], rp_ref, gp_ref, root_ref, rb_ref, gb_ref,
                        hx_ref, hs_ref, hd_ref, c_ref)
    xh2_ref[...] = xh2
    _tables_core(xh2, rw2_ref, gw2_ref, asrc2_ref, adst2_ref,
                 hr2_ref, hx2_ref, hs2_ref, hd2_ref, c2_ref)


def _combine_tables(xh, rp, gp, root, rb, gb, hx, hs, hd, c,
                    rw2, gw2, asrc2, adst2):
    return pl.pallas_call(
        _combine_tables_body,
        out_shape=(jax.ShapeDtypeStruct((N, H), jnp.float32),) + _TABLES_OUT,
    )(xh, rp, gp, root, rb, gb, hx, hs, hd, c, rw2, gw2, asrc2, adst2)


def _combine_out_body(xh_ref, rp_ref, gp_ref, root_ref, rb_ref, gb_ref,
                      hx_ref, hs_ref, hd_ref, c_ref, wout_ref, bout_ref,
                      out_ref):
    xh2 = _combine_core(xh_ref[...], rp_ref, gp_ref, root_ref, rb_ref, gb_ref,
                        hx_ref, hs_ref, hd_ref, c_ref)
    out_ref[...] = _mm(xh2, wout_ref[...]) + bout_ref[...]


def _combine_out(xh, rp, gp, root, rb, gb, hx, hs, hd, c, wout, bout):
    return pl.pallas_call(
        _combine_out_body,
        out_shape=jax.ShapeDtypeStruct((N, OUT_DIM), jnp.float32),
    )(xh, rp, gp, root, rb, gb, hx, hs, hd, c, wout, bout)


def _sc_edge_body(gidx_hbm, seg_hbm, src_hbm, dst_hbm, hr_hbm, hx_hbm,
                  hs_hbm, hd_hbm, c_hbm,
                  rgcn_out, gat_out,
                  rows0_v, rows1_v, ex_v, ia_v, ib_v, ga_v, sb_v,
                  hs_v, hd_v, c_v,
                  racc, gacc, sem0, sem1, ssem0, ssem1):
    cid = lax.axis_index("c")
    sid = lax.axis_index("s")
    wid = cid * NS + sid
    rows = (rows0_v, rows1_v)
    sems = (sem0, sem1)
    ssems = (ssem0, ssem1)

    pltpu.sync_copy(src_hbm.at[wid, :, :], ia_v)
    pltpu.sync_copy(dst_hbm.at[wid, :, :], ib_v)
    pltpu.sync_copy(gidx_hbm.at[wid, :, :], ga_v)
    pltpu.sync_copy(seg_hbm.at[wid, :, :], sb_v)
    pltpu.sync_copy(hs_hbm, hs_v)
    pltpu.sync_copy(hd_hbm, hd_v)
    pltpu.sync_copy(c_hbm, c_v)

    def zrow(i, _):
        r = i // (HE // 16)
        k = i % (HE // 16)
        rows0_v[r, pl.ds(k * 16, 16)] = jnp.zeros((16,), jnp.float32)
        return 0
    lax.fori_loop(0, CHUNK * (HE // 16), zrow, 0)

    rrows = (R * N) // NS
    grows = N // NS
    for j in range(rrows // CHUNK):
        pltpu.sync_copy(rows0_v, racc.at[pl.ds(sid * rrows + j * CHUNK, CHUNK)])
    pltpu.sync_copy(rows0_v, gacc.at[pl.ds(sid * grows, grows)])
    plsc.subcore_barrier()

    c = c_v[...][0]

    GPC = CHUNK // 16

    def score(g, _):
        gc = g // GPC
        go = g % GPC
        s16 = ia_v[gc, pl.ds(go * 16, 16)]
        d16 = ib_v[gc, pl.ds(go * 16, 16)]
        sc = plsc.load_gather(hs_v, [s16]) + plsc.load_gather(hd_v, [d16])
        ex_v[pl.ds(g * 16, 16)] = jnp.exp(_leaky(sc) - c)
        return 0
    lax.fori_loop(0, EPW // 16, score, 0)

    TOT = 2 * NCHUNK

    def issue(c, b):
        @pl.when(c < NCHUNK)
        def _():
            pltpu.async_copy(hx_hbm.at[ia_v.at[c]], rows[b], sems[b])

        @pl.when(c >= NCHUNK)
        def _():
            pltpu.async_copy(hr_hbm.at[ga_v.at[c - NCHUNK]], rows[b], sems[b])

    def wait_gather(b):
        pltpu.make_async_copy(hx_hbm.at[ia_v.at[0]], rows[b], sems[b]).wait()

    def wait_scatter(b):
        pltpu.make_async_copy(hx_hbm.at[pl.ds(0, CHUNK)], rows[b],
                              ssems[b]).wait()

    def drain_process(c, b):
        wait_gather(b)

        @pl.when(c < NCHUNK)
        def _():
            def scalegrp(g, _):
                ex16 = ex_v[pl.ds(c * CHUNK + g * 16, 16)]
                for i in range(16):
                    a = ex16[i]
                    e = g * 16 + i
                    for j in range(HE // 16):
                        sl = pl.ds(j * 16, 16)
                        rows[b][e, sl] = rows[b][e, sl] * a
                return 0
            lax.fori_loop(0, CHUNK // 16, scalegrp, 0)
            pltpu.async_copy(rows[b], gacc.at[ib_v.at[c]], ssems[b], add=True)

        @pl.when(c >= NCHUNK)
        def _():
            pltpu.async_copy(rows[b], racc.at[sb_v.at[c - NCHUNK]],
                             ssems[b], add=True)

    pltpu.async_copy(hx_hbm.at[ia_v.at[0]], rows0_v, sem0)

    def pipe(i, _):
        for b in range(2):
            c = 2 * i + b

            @pl.when(c + 1 < TOT)
            def _():
                @pl.when(c >= 1)
                def _():
                    wait_scatter(1 - b)
                issue(c + 1, 1 - b)
            drain_process(c, b)
        return 0
    lax.fori_loop(0, TOT // 2, pipe, 0)
    wait_scatter(0)
    wait_scatter(1)

    plsc.subcore_barrier()
    for j in range(rrows // CHUNK):
        off = sid * rrows + j * CHUNK
        pltpu.sync_copy(racc.at[pl.ds(off, CHUNK)],
                        rgcn_out.at[cid, pl.ds(off, CHUNK)])
    pltpu.sync_copy(gacc.at[pl.ds(sid * grows, grows)],
                    gat_out.at[cid, pl.ds(sid * grows, grows)])


@functools.lru_cache(maxsize=1)
def _build_sc_edge():
    return pl.kernel(
        _sc_edge_body,
        out_type=(
            jax.ShapeDtypeStruct((NC, R * N, HE), jnp.float32),
            jax.ShapeDtypeStruct((NC, N, HE), jnp.float32),
        ),
        mesh=plsc.VectorSubcoreMesh(core_axis_name="c", subcore_axis_name="s",
                                    num_cores=NC, num_subcores=NS),
        compiler_params=pltpu.CompilerParams(
            needs_layout_passes=False, use_tc_tiling_on_sc=False),
        scratch_types=[
            pltpu.VMEM((CHUNK, HE), jnp.float32),
            pltpu.VMEM((CHUNK, HE), jnp.float32),
            pltpu.VMEM((EPW,), jnp.float32),
            pltpu.VMEM((NCHUNK, CHUNK), jnp.int32),
            pltpu.VMEM((NCHUNK, CHUNK), jnp.int32),
            pltpu.VMEM((NCHUNK, CHUNK), jnp.int32),
            pltpu.VMEM((NCHUNK, CHUNK), jnp.int32),
            pltpu.VMEM((N,), jnp.float32),
            pltpu.VMEM((N,), jnp.float32),
            pltpu.VMEM((16,), jnp.float32),
            pltpu.VMEM_SHARED((R * N, HE), jnp.float32),
            pltpu.VMEM_SHARED((N, HE), jnp.float32),
            pltpu.SemaphoreType.DMA,
            pltpu.SemaphoreType.DMA,
            pltpu.SemaphoreType.DMA,
            pltpu.SemaphoreType.DMA,
        ],
    )


def _sc_edge(*args):
    return _build_sc_edge()(*args)


def kernel(x, edge_index, edge_type, W_in, b_in, W_tfc, b_tfc, in_proj_w,
           in_proj_b, out_proj_w, out_proj_b, rgcn0_w, rgcn0_root, rgcn0_b,
           gat0_w, gat0_att_src, gat0_att_dst, gat0_b, rgcn1_w, rgcn1_root,
           rgcn1_b, gat1_w, gat1_att_src, gat1_att_dst, gat1_b, W_out, b_out):
    src = edge_index[0].astype(jnp.int32)
    dst = edge_index[1].astype(jnp.int32)
    et = edge_type.astype(jnp.int32)
    shp = (NW, NCHUNK, CHUNK)
    gidx = (et * N + src).reshape(shp)
    seg = (et * N + dst).reshape(shp)
    src = src.reshape(shp)
    dst = dst.reshape(shp)

    wq, wk, wv = (in_proj_w[i * H:(i + 1) * H].reshape(HEADS, HEAD_DIM, H)
                  for i in range(3))
    bq, bk, bv = (in_proj_b[i * H:(i + 1) * H].reshape(HEADS, 1, HEAD_DIM)
                  for i in range(3))
    ow = out_proj_w.T.reshape(HEADS, HEAD_DIM, H)
    xh, hr, hx, hs, hd, c = _prologue(
        x, W_in, b_in.reshape(1, H), W_tfc, b_tfc.reshape(1, H),
        wq, wk, wv, bq, bk, bv, ow, out_proj_b.reshape(1, H),
        rgcn0_w, gat0_w, gat0_att_src.reshape(H, 1), gat0_att_dst.reshape(H, 1))
    rp, gp = _sc_edge(gidx, seg, src, dst, hr, hx,
                      hs.reshape(N), hd.reshape(N),
                      jnp.broadcast_to(c.reshape(1), (16,)))
    xh1, hr1, hx1, hs1, hd1, c1 = _combine_tables(
        xh, rp, gp, rgcn0_root, rgcn0_b.reshape(1, H), gat0_b.reshape(1, H),
        hx, hs, hd, c, rgcn1_w, gat1_w,
        gat1_att_src.reshape(H, 1), gat1_att_dst.reshape(H, 1))
    rp1, gp1 = _sc_edge(gidx, seg, src, dst, hr1, hx1,
                        hs1.reshape(N), hd1.reshape(N),
                        jnp.broadcast_to(c1.reshape(1), (16,)))
    out = _combine_out(xh1, rp1, gp1, rgcn1_root, rgcn1_b.reshape(1, H),
                       gat1_b.reshape(1, H), hx1, hs1, hd1, c1,
                       W_out, b_out.reshape(1, OUT_DIM))
    return out.reshape(1, N, OUT_DIM)
